# baseline (speedup 1.0000x reference)
import jax
import jax.numpy as jnp
from jax.experimental import pallas as pl
from jax.experimental.pallas import tpu as pltpu

_B = 16
_MAX_SEQ = 2048
_H = 8
_D = 128
_PAGE = 256
_T = 32
_PAGES_PER_SEQ = _MAX_SEQ // _PAGE
_NUM_PAGES = _B * _PAGES_PER_SEQ


def _body(srcb_ref, has_ref, slot_ref, k_ref, v_ref, kc_ref, vc_ref,
          ko_ref, vo_ref):
    p = pl.program_id(0)
    ko_ref[...] = kc_ref[...]
    vo_ref[...] = vc_ref[...]

    @pl.when(has_ref[p] == 1)
    def _():
        s = pl.multiple_of(slot_ref[p], 8)
        ko_ref[pl.ds(s, _T), :] = k_ref[...]
        vo_ref[pl.ds(s, _T), :] = v_ref[...]


def kernel(k, v, k_cache, v_cache, page_table, cache_seqlens):
    k2 = k.reshape(_B * _T, _H * _D)
    v2 = v.reshape(_B * _T, _H * _D)
    kc2 = k_cache.reshape(_NUM_PAGES * _PAGE, _H * _D)
    vc2 = v_cache.reshape(_NUM_PAGES * _PAGE, _H * _D)

    pos0 = cache_seqlens
    pg = pos0 // _PAGE
    tp = jnp.take_along_axis(page_table, pg[:, None], axis=1)[:, 0]
    s0 = pos0 % _PAGE
    has = jnp.zeros((_NUM_PAGES,), jnp.int32).at[tp].set(1)
    srcb = jnp.zeros((_NUM_PAGES,), jnp.int32).at[tp].set(
        jnp.arange(_B, dtype=jnp.int32))
    slot0 = jnp.zeros((_NUM_PAGES,), jnp.int32).at[tp].set(s0)

    grid_spec = pltpu.PrefetchScalarGridSpec(
        num_scalar_prefetch=3,
        grid=(_NUM_PAGES,),
        in_specs=[
            pl.BlockSpec((_T, _H * _D), lambda p, srcb, has, slot: (srcb[p], 0)),
            pl.BlockSpec((_T, _H * _D), lambda p, srcb, has, slot: (srcb[p], 0)),
            pl.BlockSpec((_PAGE, _H * _D), lambda p, srcb, has, slot: (p, 0)),
            pl.BlockSpec((_PAGE, _H * _D), lambda p, srcb, has, slot: (p, 0)),
        ],
        out_specs=[
            pl.BlockSpec((_PAGE, _H * _D), lambda p, srcb, has, slot: (p, 0)),
            pl.BlockSpec((_PAGE, _H * _D), lambda p, srcb, has, slot: (p, 0)),
        ],
    )

    ko2, vo2 = pl.pallas_call(
        _body,
        grid_spec=grid_spec,
        out_shape=[
            jax.ShapeDtypeStruct((_NUM_PAGES * _PAGE, _H * _D), k_cache.dtype),
            jax.ShapeDtypeStruct((_NUM_PAGES * _PAGE, _H * _D), v_cache.dtype),
        ],
    )(srcb, has, slot0, k2, v2, kc2, vc2)

    k_cache_new = ko2.reshape(_NUM_PAGES, _PAGE, _H, _D)
    v_cache_new = vo2.reshape(_NUM_PAGES, _PAGE, _H, _D)
    return (k_cache_new, v_cache_new, cache_seqlens + _T)
